# Initial kernel scaffold; baseline (speedup 1.0000x reference)
import jax
import jax.numpy as jnp
from jax.experimental import pallas as pl


def kernel(atomic_num, coord, abs_mask, edge_index, edge_length, W_atom, b_atom, W_coord, b_coord, W_node, b_node, exp_p, eps_layer, W_mlp, b_mlp):
    raise NotImplementedError("write your pallas kernel here")


# speedup vs baseline: 51.6950x; 51.6950x over previous
import dataclasses
import functools

import jax
import jax.numpy as jnp
from jax import lax
from jax.experimental import pallas as pl
from jax.experimental.pallas import tpu as pltpu
from jax.experimental.pallas import tpu_sc as plsc

_N = 10000
_E = 640000
_DA = 118
_DC = 3
_H = 128
_OUT = 100

_NPAD = 10240
_BN = 1024
_NB = _NPAD // _BN
_NC = 2
_NS = 16
_NW = _NC * _NS
_EPW = _E // _NW
_L = 16

_sc_mesh = plsc.VectorSubcoreMesh(core_axis_name="c", subcore_axis_name="s")
_sc_params = pltpu.CompilerParams()
if "needs_layout_passes" in pltpu.CompilerParams.__dataclass_fields__:
    _sc_params = dataclasses.replace(_sc_params, needs_layout_passes=False)


@functools.partial(
    pl.kernel,
    out_type=jax.ShapeDtypeStruct((_NB, _NW, _BN), jnp.float32),
    mesh=_sc_mesh,
    compiler_params=_sc_params,
    scratch_types=[
        pltpu.VMEM((_NPAD,), jnp.float32),
        pltpu.VMEM((_NPAD,), jnp.float32),
        pltpu.VMEM((_EPW,), jnp.int32),
        pltpu.VMEM((_EPW,), jnp.int32),
    ],
)
def _sc_counts(src_hbm, dst_hbm, maskf_hbm, out_hbm, mask_v, cnt_v, src_v, dst_v):
    wid = lax.axis_index("s") * _NC + lax.axis_index("c")
    base = wid * _EPW
    pltpu.sync_copy(maskf_hbm, mask_v)
    pltpu.sync_copy(src_hbm.at[pl.ds(base, _EPW)], src_v)
    pltpu.sync_copy(dst_hbm.at[pl.ds(base, _EPW)], dst_v)

    @pl.loop(0, _NPAD, step=_L)
    def _(i):
        cnt_v[pl.ds(i, _L)] = jnp.zeros((_L,), jnp.float32)

    @pl.loop(0, _EPW, step=_L)
    def _(e):
        d = dst_v[pl.ds(e, _L)]
        m = plsc.load_gather(mask_v, [d])
        s = src_v[pl.ds(e, _L)]
        plsc.addupdate_scatter(cnt_v, [s], m)

    @pl.loop(0, _NB)
    def _(i):
        pltpu.sync_copy(cnt_v.at[pl.ds(i * _BN, _BN)], out_hbm.at[i, wid])


def _tc_body(eps_ref, mask_ref, cnt_ref, x_ref, wbig_ref, bac_ref, wnt_ref,
             bn_ref, wmt_ref, bm_ref, out_ref, acc_ref, s_ref):
    i = pl.program_id(0)

    @pl.when(i == 0)
    def _():
        acc_ref[...] = jnp.zeros_like(acc_ref)
        s_ref[0] = 0.0

    cnt = jnp.sum(cnt_ref[0], axis=0)
    v = (1.0 + eps_ref[0]) * mask_ref[0, 0, :] + cnt
    vb = v[None, :]
    acc_ref[...] += jnp.dot(vb, x_ref[...], preferred_element_type=jnp.float32, precision=lax.Precision.HIGHEST)
    s_ref[0] += jnp.sum(v)

    @pl.when(i == _NB - 1)
    def _():
        s = s_ref[0]
        ac = jnp.dot(acc_ref[...], wbig_ref[...],
                     preferred_element_type=jnp.float32, precision=lax.Precision.HIGHEST) + s * bac_ref[...]
        f = jnp.dot(ac, wnt_ref[...], preferred_element_type=jnp.float32, precision=lax.Precision.HIGHEST) \
            + s * bn_ref[...]
        logits = jnp.dot(f * (1.0 / _N), wmt_ref[...],
                         preferred_element_type=jnp.float32, precision=lax.Precision.HIGHEST) + bm_ref[...]
        out_ref[...] = jax.nn.sigmoid(logits)


_tc_reduce = pl.pallas_call(
    _tc_body,
    grid=(_NB,),
    in_specs=[
        pl.BlockSpec(memory_space=pltpu.SMEM),
        pl.BlockSpec((1, 1, _BN), lambda i: (i, 0, 0)),
        pl.BlockSpec((1, _NW, _BN), lambda i: (i, 0, 0)),
        pl.BlockSpec((_BN, _H), lambda i: (i, 0)),
        pl.BlockSpec((_H, 2 * _H), lambda i: (0, 0)),
        pl.BlockSpec((1, 2 * _H), lambda i: (0, 0)),
        pl.BlockSpec((2 * _H, _H), lambda i: (0, 0)),
        pl.BlockSpec((1, _H), lambda i: (0, 0)),
        pl.BlockSpec((_H, _OUT), lambda i: (0, 0)),
        pl.BlockSpec((1, _OUT), lambda i: (0, 0)),
    ],
    out_specs=pl.BlockSpec((1, _OUT), lambda i: (0, 0)),
    out_shape=jax.ShapeDtypeStruct((1, _OUT), jnp.float32),
    scratch_shapes=[
        pltpu.VMEM((1, _H), jnp.float32),
        pltpu.SMEM((1,), jnp.float32),
    ],
)


def kernel(atomic_num, coord, abs_mask, edge_index, edge_length,
           W_atom, b_atom, W_coord, b_coord, W_node, b_node,
           exp_p, eps_layer, W_mlp, b_mlp):
    del edge_length, exp_p
    maskf = jnp.zeros((_NPAD,), jnp.float32).at[:_N].set(
        (abs_mask != 0).astype(jnp.float32))
    cnt3 = _sc_counts(edge_index[0], edge_index[1], maskf)

    x = jnp.zeros((_NPAD, _H), jnp.float32)
    x = x.at[:_N, :_DA].set(atomic_num).at[:_N, _DA:_DA + _DC].set(coord)
    wbig = jnp.zeros((_H, 2 * _H), jnp.float32)
    wbig = wbig.at[:_DA, :_H].set(W_atom.T)
    wbig = wbig.at[_DA:_DA + _DC, _H:].set(W_coord.T)
    bac = jnp.concatenate([b_atom, b_coord])[None, :]

    return _tc_reduce(
        eps_layer,
        maskf.reshape(_NB, 1, _BN),
        cnt3,
        x,
        wbig,
        bac,
        W_node.T,
        b_node[None, :],
        W_mlp.T,
        b_mlp[None, :],
    )
